# baseline (speedup 1.0000x reference)
import functools

import jax
import jax.numpy as jnp
from jax import lax
from jax.experimental import pallas as pl
from jax.experimental.pallas import tpu as pltpu
from jax.experimental.pallas import tpu_sc as plsc

NC, NS = 2, 16
NW = NC * NS
EB = 256
VL = 16


def _cdiv(a, b):
    return (a + b - 1) // b


def _make_gather(n_rows_pad, d):
    b_per_w = n_rows_pad // NW
    mesh = plsc.VectorSubcoreMesh(core_axis_name="c", subcore_axis_name="s")

    @functools.partial(
        pl.kernel,
        out_type=jax.ShapeDtypeStruct((n_rows_pad, d), jnp.float32),
        mesh=mesh,
        scratch_types=[
            pltpu.VMEM((b_per_w,), jnp.int32),
            pltpu.VMEM((b_per_w, d), jnp.float32),
            pltpu.SemaphoreType.DMA,
        ],
    )
    def gather_kernel(table_hbm, idx_hbm, out_hbm, idx_v, rows_v, sem):
        wid = lax.axis_index("s") * NC + lax.axis_index("c")
        base = wid * b_per_w
        pltpu.sync_copy(idx_hbm.at[pl.ds(base, b_per_w)], idx_v)
        pltpu.async_copy(table_hbm.at[idx_v], rows_v, sem).wait()
        pltpu.sync_copy(rows_v, out_hbm.at[pl.ds(base, b_per_w)])

    return gather_kernel


def _make_edge_agg(n_acc, d, k_blocks):
    mesh = plsc.VectorSubcoreMesh(core_axis_name="c", subcore_axis_name="s")
    rows_z = n_acc // NS

    @functools.partial(
        pl.kernel,
        out_type=jax.ShapeDtypeStruct((NC, n_acc, d), jnp.float32),
        mesh=mesh,
        scratch_types=[
            pltpu.VMEM((EB,), jnp.int32),
            pltpu.VMEM((EB,), jnp.int32),
            pltpu.VMEM((EB, d), jnp.float32),
            pltpu.VMEM_SHARED((n_acc, d), jnp.float32),
            pltpu.SemaphoreType.DMA,
        ],
    )
    def edge_agg(g_hbm, src_hbm, dst_hbm, zg_hbm, pg_hbm,
                 src_blk, dst_blk, rows_v, acc, sem):
        cid = lax.axis_index("c")
        sid = lax.axis_index("s")
        wid = sid * NC + cid

        pltpu.sync_copy(zg_hbm.at[pl.ds(sid * rows_z, rows_z)],
                        acc.at[pl.ds(sid * rows_z, rows_z)])
        plsc.subcore_barrier()

        @pl.loop(0, k_blocks)
        def _(j):
            base = (wid * k_blocks + j) * EB
            pltpu.sync_copy(src_hbm.at[pl.ds(base, EB)], src_blk)
            pltpu.sync_copy(dst_hbm.at[pl.ds(base, EB)], dst_blk)
            pltpu.async_copy(g_hbm.at[src_blk], rows_v, sem).wait()
            pltpu.sync_copy(rows_v, acc.at[dst_blk], add=True)

        plsc.subcore_barrier()
        pltpu.sync_copy(acc.at[pl.ds(sid * rows_z, rows_z)],
                        pg_hbm.at[cid].at[pl.ds(sid * rows_z, rows_z)])

    return edge_agg


def _make_counts(n_acc, d, k_blocks):
    mesh = plsc.VectorSubcoreMesh(core_axis_name="c", subcore_axis_name="s")
    rows_z = n_acc // NS

    @functools.partial(
        pl.kernel,
        out_type=jax.ShapeDtypeStruct((NC, n_acc, d), jnp.float32),
        mesh=mesh,
        scratch_types=[
            pltpu.VMEM((EB,), jnp.int32),
            pltpu.VMEM((EB, d), jnp.float32),
            pltpu.VMEM_SHARED((n_acc, d), jnp.float32),
        ],
    )
    def counts_kernel(dst_hbm, zc_hbm, ones_hbm, out_hbm,
                      dst_blk, ones_v, acc):
        cid = lax.axis_index("c")
        sid = lax.axis_index("s")
        wid = sid * NC + cid

        pltpu.sync_copy(zc_hbm.at[pl.ds(sid * rows_z, rows_z)],
                        acc.at[pl.ds(sid * rows_z, rows_z)])
        pltpu.sync_copy(ones_hbm, ones_v)
        plsc.subcore_barrier()

        @pl.loop(0, k_blocks)
        def _(j):
            base = (wid * k_blocks + j) * EB
            pltpu.sync_copy(dst_hbm.at[pl.ds(base, EB)], dst_blk)
            pltpu.sync_copy(ones_v, acc.at[dst_blk], add=True)

        plsc.subcore_barrier()
        pltpu.sync_copy(acc.at[pl.ds(sid * rows_z, rows_z)],
                        out_hbm.at[cid].at[pl.ds(sid * rows_z, rows_z)])

    return counts_kernel


def _make_combine(n_rows, d, rb):
    grid = n_rows // rb

    def combine_body(pg_ref, pc_ref, h_ref, wl_ref, b_ref, wr_ref, out_ref):
        agg = pg_ref[0] + pg_ref[1]
        cnt = pc_ref[0, :, 0:1] + pc_ref[1, :, 0:1]
        inv = 1.0 / jnp.maximum(cnt, 1.0)
        mean = agg * inv
        out = (jnp.dot(mean, wl_ref[...], preferred_element_type=jnp.float32)
               + b_ref[...]
               + jnp.dot(h_ref[...], wr_ref[...],
                         preferred_element_type=jnp.float32))
        out_ref[...] = jnp.maximum(out, 0.0)

    return pl.pallas_call(
        combine_body,
        grid=(grid,),
        in_specs=[
            pl.BlockSpec((NC, rb, d), lambda i: (0, i, 0)),
            pl.BlockSpec((NC, rb, d), lambda i: (0, i, 0)),
            pl.BlockSpec((rb, d), lambda i: (i, 0)),
            pl.BlockSpec((d, d), lambda i: (0, 0)),
            pl.BlockSpec((1, d), lambda i: (0, 0)),
            pl.BlockSpec((d, d), lambda i: (0, 0)),
        ],
        out_specs=pl.BlockSpec((rb, d), lambda i: (i, 0)),
        out_shape=jax.ShapeDtypeStruct((n_rows, d), jnp.float32),
    )


def kernel(x, edge_index, table, W1l, b1l, W1r, W2l, b2l, W2r):
    n = x.shape[0]
    e = edge_index.shape[1]
    d = table.shape[1]

    rb = 512
    P = _cdiv(n + 1, 2560) * 2560
    xp = jnp.concatenate([x, jnp.zeros((P - n,), jnp.int32)])
    h0 = _make_gather(P, d)(table, xp)

    k_blocks = _cdiv(e, NW * EB)
    e_pad = NW * k_blocks * EB
    src = edge_index[0]
    dst = edge_index[1]
    if e_pad != e:
        src = jnp.concatenate([src, jnp.zeros((e_pad - e,), jnp.int32)])
        dst = jnp.concatenate([dst, jnp.full((e_pad - e,), n, jnp.int32)])

    zeros_g = jnp.zeros((P, d), jnp.float32)
    zeros_c = jnp.zeros((P, d), jnp.float32)

    edge_agg = _make_edge_agg(P, d, k_blocks)
    combine = _make_combine(P, d, rb)

    b1 = b1l.reshape(1, d)
    b2 = b2l.reshape(1, d)

    ones_e = jnp.ones((EB, d), jnp.float32)
    pc = _make_counts(P, d, k_blocks)(dst, zeros_c, ones_e)
    pg1 = edge_agg(h0, src, dst, zeros_g)
    h1 = combine(pg1, pc, h0, W1l, b1, W1r)
    pg2 = edge_agg(h1, src, dst, zeros_g)
    h2 = combine(pg2, pc, h1, W2l, b2, W2r)
    return h2[:n]


# speedup vs baseline: 1.1860x; 1.1860x over previous
import functools

import jax
import jax.numpy as jnp
from jax import lax
from jax.experimental import pallas as pl
from jax.experimental.pallas import tpu as pltpu
from jax.experimental.pallas import tpu_sc as plsc

NC, NS = 2, 16
NW = NC * NS
EB = 128
VL = 16


def _cdiv(a, b):
    return (a + b - 1) // b


def _make_gather(n_rows_pad, d):
    b_per_w = n_rows_pad // NW
    mesh = plsc.VectorSubcoreMesh(core_axis_name="c", subcore_axis_name="s")

    @functools.partial(
        pl.kernel,
        out_type=jax.ShapeDtypeStruct((n_rows_pad, d), jnp.float32),
        mesh=mesh,
        scratch_types=[
            pltpu.VMEM((b_per_w,), jnp.int32),
            pltpu.VMEM((b_per_w, d), jnp.float32),
            pltpu.SemaphoreType.DMA,
        ],
    )
    def gather_kernel(table_hbm, idx_hbm, out_hbm, idx_v, rows_v, sem):
        wid = lax.axis_index("s") * NC + lax.axis_index("c")
        base = wid * b_per_w
        pltpu.sync_copy(idx_hbm.at[pl.ds(base, b_per_w)], idx_v)
        pltpu.async_copy(table_hbm.at[idx_v], rows_v, sem).wait()
        pltpu.sync_copy(rows_v, out_hbm.at[pl.ds(base, b_per_w)])

    return gather_kernel


def _make_edge_agg(n_acc, d, k_blocks):
    mesh = plsc.VectorSubcoreMesh(core_axis_name="c", subcore_axis_name="s")
    rows_z = n_acc // NS

    @functools.partial(
        pl.kernel,
        out_type=jax.ShapeDtypeStruct((NC, n_acc, d), jnp.float32),
        mesh=mesh,
        scratch_types=[
            pltpu.VMEM((EB,), jnp.int32),
            pltpu.VMEM((EB,), jnp.int32),
            pltpu.VMEM((EB, d), jnp.float32),
            pltpu.VMEM_SHARED((n_acc, d), jnp.float32),
            pltpu.SemaphoreType.DMA,
        ],
    )
    def edge_agg(g_hbm, src_hbm, dst_hbm, zg_hbm, pg_hbm,
                 src_blk, dst_blk, rows_v, acc, sem):
        cid = lax.axis_index("c")
        sid = lax.axis_index("s")
        wid = sid * NC + cid

        pltpu.sync_copy(zg_hbm.at[pl.ds(sid * rows_z, rows_z)],
                        acc.at[pl.ds(sid * rows_z, rows_z)])
        plsc.subcore_barrier()

        @pl.loop(0, k_blocks)
        def _(j):
            base = (wid * k_blocks + j) * EB
            pltpu.sync_copy(src_hbm.at[pl.ds(base, EB)], src_blk)
            pltpu.sync_copy(dst_hbm.at[pl.ds(base, EB)], dst_blk)
            pltpu.async_copy(g_hbm.at[src_blk], rows_v, sem).wait()
            pltpu.sync_copy(rows_v, acc.at[dst_blk], add=True)

        plsc.subcore_barrier()
        pltpu.sync_copy(acc.at[pl.ds(sid * rows_z, rows_z)],
                        pg_hbm.at[cid].at[pl.ds(sid * rows_z, rows_z)])

    return edge_agg


def _make_counts(n_acc, d, k_blocks):
    mesh = plsc.VectorSubcoreMesh(core_axis_name="c", subcore_axis_name="s")
    rows_z = n_acc // NS

    @functools.partial(
        pl.kernel,
        out_type=jax.ShapeDtypeStruct((NC, n_acc, d), jnp.float32),
        mesh=mesh,
        scratch_types=[
            pltpu.VMEM((EB,), jnp.int32),
            pltpu.VMEM((EB, d), jnp.float32),
            pltpu.VMEM_SHARED((n_acc, d), jnp.float32),
        ],
    )
    def counts_kernel(dst_hbm, zc_hbm, ones_hbm, out_hbm,
                      dst_blk, ones_v, acc):
        cid = lax.axis_index("c")
        sid = lax.axis_index("s")
        wid = sid * NC + cid

        pltpu.sync_copy(zc_hbm.at[pl.ds(sid * rows_z, rows_z)],
                        acc.at[pl.ds(sid * rows_z, rows_z)])
        pltpu.sync_copy(ones_hbm, ones_v)
        plsc.subcore_barrier()

        @pl.loop(0, k_blocks)
        def _(j):
            base = (wid * k_blocks + j) * EB
            pltpu.sync_copy(dst_hbm.at[pl.ds(base, EB)], dst_blk)
            pltpu.sync_copy(ones_v, acc.at[dst_blk], add=True)

        plsc.subcore_barrier()
        pltpu.sync_copy(acc.at[pl.ds(sid * rows_z, rows_z)],
                        out_hbm.at[cid].at[pl.ds(sid * rows_z, rows_z)])

    return counts_kernel


def _make_combine(n_rows, d, rb):
    grid = n_rows // rb

    def combine_body(pg_ref, pc_ref, h_ref, wl_ref, b_ref, wr_ref, out_ref):
        agg = pg_ref[0] + pg_ref[1]
        cnt = pc_ref[0, :, 0:1] + pc_ref[1, :, 0:1]
        inv = 1.0 / jnp.maximum(cnt, 1.0)
        mean = agg * inv
        out = (jnp.dot(mean, wl_ref[...], preferred_element_type=jnp.float32)
               + b_ref[...]
               + jnp.dot(h_ref[...], wr_ref[...],
                         preferred_element_type=jnp.float32))
        out_ref[...] = jnp.maximum(out, 0.0)

    return pl.pallas_call(
        combine_body,
        grid=(grid,),
        in_specs=[
            pl.BlockSpec((NC, rb, d), lambda i: (0, i, 0)),
            pl.BlockSpec((NC, rb, d), lambda i: (0, i, 0)),
            pl.BlockSpec((rb, d), lambda i: (i, 0)),
            pl.BlockSpec((d, d), lambda i: (0, 0)),
            pl.BlockSpec((1, d), lambda i: (0, 0)),
            pl.BlockSpec((d, d), lambda i: (0, 0)),
        ],
        out_specs=pl.BlockSpec((rb, d), lambda i: (i, 0)),
        out_shape=jax.ShapeDtypeStruct((n_rows, d), jnp.float32),
    )


def kernel(x, edge_index, table, W1l, b1l, W1r, W2l, b2l, W2r):
    n = x.shape[0]
    e = edge_index.shape[1]
    d = table.shape[1]

    rb = 512
    P = _cdiv(n + 1, 2560) * 2560
    xp = jnp.concatenate([x, jnp.zeros((P - n,), jnp.int32)])
    h0 = _make_gather(P, d)(table, xp)

    k_blocks = _cdiv(e, NW * EB)
    e_pad = NW * k_blocks * EB
    src = edge_index[0]
    dst = edge_index[1]
    if e_pad != e:
        src = jnp.concatenate([src, jnp.zeros((e_pad - e,), jnp.int32)])
        dst = jnp.concatenate([dst, jnp.full((e_pad - e,), n, jnp.int32)])

    zeros_g = jnp.zeros((P, d), jnp.float32)
    zeros_c = jnp.zeros((P, d), jnp.float32)

    edge_agg = _make_edge_agg(P, d, k_blocks)
    combine = _make_combine(P, d, rb)

    b1 = b1l.reshape(1, d)
    b2 = b2l.reshape(1, d)

    ones_e = jnp.ones((EB, d), jnp.float32)
    pc = _make_counts(P, d, k_blocks)(dst, zeros_c, ones_e)
    pg1 = edge_agg(h0, src, dst, zeros_g)
    h1 = combine(pg1, pc, h0, W1l, b1, W1r)
    pg2 = edge_agg(h1, src, dst, zeros_g)
    h2 = combine(pg2, pc, h1, W2l, b2, W2r)
    return h2[:n]
